# baseline (speedup 1.0000x reference)
import functools

import jax
import jax.numpy as jnp
from jax import lax
from jax.experimental import pallas as pl
from jax.experimental.pallas import tpu as pltpu
from jax.experimental.pallas import tpu_sc as plsc

N = 10000
E = 320000
D = 128
NC = 2
NS = 16
NW = NC * NS
CHUNK = 128
NBUF = 2
C_PER_W = 80
IDXB = 16
NBLK = C_PER_W // IDXB
E_PAD = NW * C_PER_W * CHUNK
N_CHUNKS = E_PAD // CHUNK
C_PER_TILE = N_CHUNKS // NS
N_PAD = 10240
ROWS_PER_TILE = N_PAD // NS

_mesh = plsc.VectorSubcoreMesh(core_axis_name="c", subcore_axis_name="s")


def _deg_body(idxs, ones_hbm, zeros_hbm, degp, idx_v, ones_v, acc):
    c = lax.axis_index("c")
    s = lax.axis_index("s")
    row0 = s * ROWS_PER_TILE
    pltpu.sync_copy(zeros_hbm, acc.at[pl.ds(row0, ROWS_PER_TILE)])
    pltpu.sync_copy(ones_hbm, ones_v)
    pltpu.sync_copy(idxs.at[c, pl.ds(s * C_PER_TILE, C_PER_TILE)], idx_v)
    plsc.subcore_barrier()

    def body(j, carry):
        pltpu.sync_copy(ones_v, acc.at[idx_v.at[j]], add=True)
        return carry

    lax.fori_loop(0, C_PER_TILE, body, 0)
    plsc.subcore_barrier()
    sl = pl.ds(row0, ROWS_PER_TILE)
    pltpu.sync_copy(acc.at[sl], degp.at[c, sl])


_deg_kernel = pl.kernel(
    _deg_body,
    out_type=jax.ShapeDtypeStruct((NC, N_PAD, D), jnp.float32),
    mesh=_mesh,
    scratch_types=[
        pltpu.VMEM((C_PER_TILE, CHUNK), jnp.int32),
        pltpu.VMEM((CHUNK, D), jnp.float32),
        pltpu.VMEM_SHARED((N_PAD, D), jnp.float32),
    ],
)


def _edge_body(m_hbm, src3, dst3, zeros_hbm, part, *scr):
    sblk = scr[0:2]
    dblk = scr[2:4]
    rows = scr[4:4 + NBUF]
    acc = scr[4 + NBUF]
    sems = scr[5 + NBUF:]
    c = lax.axis_index("c")
    s = lax.axis_index("s")
    w = c * NS + s
    row0 = s * ROWS_PER_TILE
    pltpu.sync_copy(zeros_hbm, acc.at[pl.ds(row0, ROWS_PER_TILE)])
    plsc.subcore_barrier()

    def load_blk(buf, k):
        sl = pl.ds(k * IDXB, IDXB)
        pltpu.sync_copy(src3.at[w, sl], sblk[buf])
        pltpu.sync_copy(dst3.at[w, sl], dblk[buf])

    def g_desc(b, j):
        kb = (j // IDXB) % 2
        return pltpu.make_async_copy(
            m_hbm.at[sblk[kb].at[j % IDXB]], rows[b], sems[b])

    load_blk(0, 0)
    if NBLK > 1:
        load_blk(1, 1)
    for b in range(NBUF):
        g_desc(b, b).start()

    for j in range(C_PER_W):
        k = j // IDXB
        if j % IDXB == 0 and j >= IDXB and k + 1 < NBLK:
            load_blk((k + 1) % 2, k + 1)
        b = j % NBUF
        g_desc(b, j).wait()
        pltpu.sync_copy(rows[b], acc.at[dblk[k % 2].at[j % IDXB]], add=True)
        if j + NBUF < C_PER_W:
            g_desc(b, j + NBUF).start()

    plsc.subcore_barrier()
    sl = pl.ds(row0, ROWS_PER_TILE)
    pltpu.sync_copy(acc.at[sl], part.at[c, sl])


_edge_kernel = pl.kernel(
    _edge_body,
    out_type=jax.ShapeDtypeStruct((NC, N_PAD, D), jnp.float32),
    mesh=_mesh,
    scratch_types=[pltpu.VMEM((IDXB, CHUNK), jnp.int32)] * 4
    + [pltpu.VMEM((CHUNK, D), jnp.float32)] * NBUF + [
        pltpu.VMEM_SHARED((N_PAD, D), jnp.float32),
    ] + [pltpu.SemaphoreType.DMA] * NBUF,
)


def _norm_body(degp_ref, x_ref, onorm_ref, inorm_ref, m0_ref):
    ds_ = degp_ref[0, :, 0:1]
    dd_ = degp_ref[1, :, 0:1]
    mask = (lax.broadcasted_iota(jnp.int32, (N_PAD, 1), 0) < N).astype(
        jnp.float32)
    onorm = lax.rsqrt(jnp.maximum(ds_, 1.0)) * mask
    inorm = lax.rsqrt(jnp.maximum(dd_, 1.0))
    onorm_ref[...] = onorm
    inorm_ref[...] = inorm
    m0_ref[...] = x_ref[...] * onorm


def _norm_kernel(degp, x_pad):
    return pl.pallas_call(
        _norm_body,
        out_shape=(
            jax.ShapeDtypeStruct((N_PAD, 1), jnp.float32),
            jax.ShapeDtypeStruct((N_PAD, 1), jnp.float32),
            jax.ShapeDtypeStruct((N_PAD, D), jnp.float32),
        ),
    )(degp, x_pad)


def _layer_body(p_ref, inorm_ref, onorm_ref, w_ref, b_ref, mnext_ref):
    agg = (p_ref[0] + p_ref[1]) * inorm_ref[...]
    z = jnp.dot(agg, w_ref[...], preferred_element_type=jnp.float32)
    h = jnp.maximum(z + b_ref[...], 0.0)
    mnext_ref[...] = h * onorm_ref[...]


def _layer_kernel(part, inorm, onorm, w, b):
    return pl.pallas_call(
        _layer_body,
        out_shape=jax.ShapeDtypeStruct((N_PAD, D), jnp.float32),
    )(part, inorm, onorm, w, b)


def _final_body(p_ref, inorm_ref, w_ref, b_ref, wc_ref, bc_ref, out_ref):
    agg = (p_ref[0] + p_ref[1]) * inorm_ref[...]
    z = jnp.dot(agg, w_ref[...], preferred_element_type=jnp.float32)
    mask = (lax.broadcasted_iota(jnp.int32, (N_PAD, 1), 0) < N).astype(
        jnp.float32)
    h = jnp.maximum(z + b_ref[...], 0.0) * mask
    hg = jnp.sum(h, axis=0, keepdims=True) * (1.0 / N)
    out_ref[...] = (
        jnp.dot(hg, wc_ref[...], preferred_element_type=jnp.float32)
        + bc_ref[...])


def _final_kernel(part, inorm, w, b, wc, bc):
    return pl.pallas_call(
        _final_body,
        out_shape=jax.ShapeDtypeStruct((1, 2), jnp.float32),
    )(part, inorm, w, b, wc, bc)


@jax.jit
def kernel(x, edge_index, W0, b0, W1, b1, W2, b2, Wc, bc):
    pad = jnp.full((E_PAD - E,), N, dtype=jnp.int32)
    src3 = jnp.concatenate([edge_index[0], pad]).reshape(NW, C_PER_W, CHUNK)
    dst3 = jnp.concatenate([edge_index[1], pad]).reshape(NW, C_PER_W, CHUNK)
    x_pad = jnp.concatenate(
        [x, jnp.zeros((N_PAD - N, D), jnp.float32)], axis=0)
    idxs = jnp.stack([src3.reshape(N_CHUNKS, CHUNK),
                      dst3.reshape(N_CHUNKS, CHUNK)])
    ones_rows = jnp.ones((CHUNK, D), jnp.float32)
    zeros_row = jnp.zeros((ROWS_PER_TILE, D), jnp.float32)

    degp = _deg_kernel(idxs, ones_rows, zeros_row)
    onorm, inorm, m = _norm_kernel(degp, x_pad)

    for (w, b) in ((W0, b0), (W1, b1)):
        part = _edge_kernel(m, src3, dst3, zeros_row)
        m = _layer_kernel(part, inorm, onorm, w, b.reshape(1, D))

    part = _edge_kernel(m, src3, dst3, zeros_row)
    return _final_kernel(part, inorm, W2, b2.reshape(1, D), Wc,
                         bc.reshape(1, 2))


# speedup vs baseline: 2.9313x; 2.9313x over previous
import functools

import jax
import jax.numpy as jnp
from jax import lax
from jax.experimental import pallas as pl
from jax.experimental.pallas import tpu as pltpu
from jax.experimental.pallas import tpu_sc as plsc

N = 10000
E = 320000
D = 128
NC = 2
NS = 16
NW = NC * NS
CHUNK = 128
NBUF = 2
C_PER_W = 80
IDXB = 16
NBLK = C_PER_W // IDXB
E_PAD = NW * C_PER_W * CHUNK
N_CHUNKS = E_PAD // CHUNK
C_PER_TILE = N_CHUNKS // NS
N_PAD = 10240
ROWS_PER_TILE = N_PAD // NS

_mesh = plsc.VectorSubcoreMesh(core_axis_name="c", subcore_axis_name="s")


def _deg_body(idxs, ones_hbm, zeros_hbm, degp, idx_v, ones_v, acc):
    c = lax.axis_index("c")
    s = lax.axis_index("s")
    row0 = s * ROWS_PER_TILE
    pltpu.sync_copy(zeros_hbm, acc.at[pl.ds(row0, ROWS_PER_TILE)])
    pltpu.sync_copy(ones_hbm, ones_v)
    pltpu.sync_copy(idxs.at[c, pl.ds(s * C_PER_TILE, C_PER_TILE)], idx_v)
    plsc.subcore_barrier()

    def body(j, carry):
        pltpu.sync_copy(ones_v, acc.at[idx_v.at[j]], add=True)
        return carry

    lax.fori_loop(0, C_PER_TILE, body, 0)
    plsc.subcore_barrier()
    sl = pl.ds(row0, ROWS_PER_TILE)
    pltpu.sync_copy(acc.at[sl], degp.at[c, sl])


_deg_kernel = pl.kernel(
    _deg_body,
    out_type=jax.ShapeDtypeStruct((NC, N_PAD, D), jnp.float32),
    mesh=_mesh,
    scratch_types=[
        pltpu.VMEM((C_PER_TILE, CHUNK), jnp.int32),
        pltpu.VMEM((CHUNK, D), jnp.float32),
        pltpu.VMEM_SHARED((N_PAD, D), jnp.float32),
    ],
)


def _edge_body(m_hbm, src3, dst3, zeros_hbm, part, *scr):
    sblk = scr[0:2]
    dblk = scr[2:4]
    rows = scr[4:4 + NBUF]
    acc = scr[4 + NBUF]
    sems = scr[5 + NBUF:]
    c = lax.axis_index("c")
    s = lax.axis_index("s")
    w = c * NS + s
    row0 = s * ROWS_PER_TILE
    pltpu.sync_copy(zeros_hbm, acc.at[pl.ds(row0, ROWS_PER_TILE)])
    plsc.subcore_barrier()

    def load_blk(buf, k):
        sl = pl.ds(k * IDXB, IDXB)
        pltpu.sync_copy(src3.at[w, sl], sblk[buf])
        pltpu.sync_copy(dst3.at[w, sl], dblk[buf])

    def g_desc(b, j):
        kb = (j // IDXB) % 2
        return pltpu.make_async_copy(
            m_hbm.at[sblk[kb].at[j % IDXB]], rows[b], sems[b])

    load_blk(0, 0)
    if NBLK > 1:
        load_blk(1, 1)
    for b in range(NBUF):
        g_desc(b, b).start()

    for j in range(C_PER_W):
        k = j // IDXB
        if j % IDXB == 0 and j >= IDXB and k + 1 < NBLK:
            load_blk((k + 1) % 2, k + 1)
        b = j % NBUF
        g_desc(b, j).wait()
        pltpu.sync_copy(rows[b], acc.at[dblk[k % 2].at[j % IDXB]], add=True)
        if j + NBUF < C_PER_W:
            g_desc(b, j + NBUF).start()

    plsc.subcore_barrier()
    sl = pl.ds(row0, ROWS_PER_TILE)
    pltpu.sync_copy(acc.at[sl], part.at[c, sl])


_edge_kernel = pl.kernel(
    _edge_body,
    out_type=jax.ShapeDtypeStruct((NC, N_PAD, D), jnp.float32),
    mesh=_mesh,
    scratch_types=[pltpu.VMEM((IDXB, CHUNK), jnp.int32)] * 4
    + [pltpu.VMEM((CHUNK, D), jnp.float32)] * NBUF + [
        pltpu.VMEM_SHARED((N_PAD, D), jnp.float32),
    ] + [pltpu.SemaphoreType.DMA] * NBUF,
)


def _norm_body(degp_ref, x_ref, onorm_ref, inorm_ref, m0_ref):
    ds_ = degp_ref[0, :, 0:1]
    dd_ = degp_ref[1, :, 0:1]
    mask = (lax.broadcasted_iota(jnp.int32, (N_PAD, 1), 0) < N).astype(
        jnp.float32)
    onorm = lax.rsqrt(jnp.maximum(ds_, 1.0)) * mask
    inorm = lax.rsqrt(jnp.maximum(dd_, 1.0))
    onorm_ref[...] = onorm
    inorm_ref[...] = inorm
    m0_ref[...] = x_ref[...] * onorm


def _norm_kernel(degp, x_pad):
    return pl.pallas_call(
        _norm_body,
        out_shape=(
            jax.ShapeDtypeStruct((N_PAD, 1), jnp.float32),
            jax.ShapeDtypeStruct((N_PAD, 1), jnp.float32),
            jax.ShapeDtypeStruct((N_PAD, D), jnp.float32),
        ),
    )(degp, x_pad)


def _layer_body(p_ref, inorm_ref, onorm_ref, w_ref, b_ref, mnext_ref):
    agg = (p_ref[0] + p_ref[1]) * inorm_ref[...]
    z = jnp.dot(agg, w_ref[...], preferred_element_type=jnp.float32)
    h = jnp.maximum(z + b_ref[...], 0.0)
    mnext_ref[...] = h * onorm_ref[...]


def _layer_kernel(part, inorm, onorm, w, b):
    return pl.pallas_call(
        _layer_body,
        out_shape=jax.ShapeDtypeStruct((N_PAD, D), jnp.float32),
    )(part, inorm, onorm, w, b)


def _final_body(p_ref, inorm_ref, w_ref, b_ref, wc_ref, bc_ref, out_ref):
    agg = (p_ref[0] + p_ref[1]) * inorm_ref[...]
    z = jnp.dot(agg, w_ref[...], preferred_element_type=jnp.float32)
    mask = (lax.broadcasted_iota(jnp.int32, (N_PAD, 1), 0) < N).astype(
        jnp.float32)
    h = jnp.maximum(z + b_ref[...], 0.0) * mask
    hg = jnp.sum(h, axis=0, keepdims=True) * (1.0 / N)
    out_ref[...] = (
        jnp.dot(hg, wc_ref[...], preferred_element_type=jnp.float32)
        + bc_ref[...])


def _final_kernel(part, inorm, w, b, wc, bc):
    return pl.pallas_call(
        _final_body,
        out_shape=jax.ShapeDtypeStruct((1, 2), jnp.float32),
    )(part, inorm, w, b, wc, bc)


@jax.jit
def kernel(x, edge_index, W0, b0, W1, b1, W2, b2, Wc, bc):
    pad = N + (jnp.arange(E_PAD - E, dtype=jnp.int32) % (N_PAD - N))
    src3 = jnp.concatenate([edge_index[0], pad]).reshape(NW, C_PER_W, CHUNK)
    dst3 = jnp.concatenate([edge_index[1], pad]).reshape(NW, C_PER_W, CHUNK)
    x_pad = jnp.concatenate(
        [x, jnp.zeros((N_PAD - N, D), jnp.float32)], axis=0)
    idxs = jnp.stack([src3.reshape(N_CHUNKS, CHUNK),
                      dst3.reshape(N_CHUNKS, CHUNK)])
    ones_rows = jnp.ones((CHUNK, D), jnp.float32)
    zeros_row = jnp.zeros((ROWS_PER_TILE, D), jnp.float32)

    degp = _deg_kernel(idxs, ones_rows, zeros_row)
    onorm, inorm, m = _norm_kernel(degp, x_pad)

    for (w, b) in ((W0, b0), (W1, b1)):
        part = _edge_kernel(m, src3, dst3, zeros_row)
        m = _layer_kernel(part, inorm, onorm, w, b.reshape(1, D))

    part = _edge_kernel(m, src3, dst3, zeros_row)
    return _final_kernel(part, inorm, W2, b2.reshape(1, D), Wc,
                         bc.reshape(1, 2))


# speedup vs baseline: 2.9461x; 1.0050x over previous
import functools

import jax
import jax.numpy as jnp
from jax import lax
from jax.experimental import pallas as pl
from jax.experimental.pallas import tpu as pltpu
from jax.experimental.pallas import tpu_sc as plsc

N = 10000
E = 320000
D = 128
NC = 2
NS = 16
NW = NC * NS
CHUNK = 128
NBUF = 2
C_PER_W = 80
IDXB = 16
NBLK = C_PER_W // IDXB
E_PAD = NW * C_PER_W * CHUNK
N_CHUNKS = E_PAD // CHUNK
C_PER_TILE = N_CHUNKS // NS
N_PAD = 10240
ROWS_PER_TILE = N_PAD // NS

_mesh = plsc.VectorSubcoreMesh(core_axis_name="c", subcore_axis_name="s")


def _deg_body(idxs, ones_hbm, zeros_hbm, degp, idx_v, ones_v, acc,
              sem0, sem1):
    sems = (sem0, sem1)
    c = lax.axis_index("c")
    s = lax.axis_index("s")
    row0 = s * ROWS_PER_TILE
    pltpu.sync_copy(zeros_hbm, acc.at[pl.ds(row0, ROWS_PER_TILE)])
    pltpu.sync_copy(ones_hbm, ones_v)
    pltpu.sync_copy(idxs.at[c, pl.ds(s * C_PER_TILE, C_PER_TILE)], idx_v)
    plsc.subcore_barrier()

    def s_desc(b, j):
        return pltpu.make_async_copy(ones_v, acc.at[idx_v.at[j]], sems[b])

    for b in range(2):
        s_desc(b, b).start(add=True)

    def body(g, carry):
        j = 2 * g
        for b in range(2):
            s_desc(b, j + b).wait()
            s_desc(b, j + b + 2).start(add=True)
        return carry

    lax.fori_loop(0, C_PER_TILE // 2 - 1, body, 0)
    for b in range(2):
        s_desc(b, C_PER_TILE - 2 + b).wait()
    plsc.subcore_barrier()
    sl = pl.ds(row0, ROWS_PER_TILE)
    pltpu.sync_copy(acc.at[sl], degp.at[c, sl])


_deg_kernel = pl.kernel(
    _deg_body,
    out_type=jax.ShapeDtypeStruct((NC, N_PAD, D), jnp.float32),
    mesh=_mesh,
    scratch_types=[
        pltpu.VMEM((C_PER_TILE, CHUNK), jnp.int32),
        pltpu.VMEM((CHUNK, D), jnp.float32),
        pltpu.VMEM_SHARED((N_PAD, D), jnp.float32),
        pltpu.SemaphoreType.DMA,
        pltpu.SemaphoreType.DMA,
    ],
)


def _edge_body(m_hbm, src3, dst3, zeros_hbm, part, *scr):
    sblk = scr[0:2]
    dblk = scr[2:4]
    rows = scr[4:4 + NBUF]
    acc = scr[4 + NBUF]
    sems = scr[5 + NBUF:]
    c = lax.axis_index("c")
    s = lax.axis_index("s")
    w = c * NS + s
    row0 = s * ROWS_PER_TILE
    pltpu.sync_copy(zeros_hbm, acc.at[pl.ds(row0, ROWS_PER_TILE)])
    plsc.subcore_barrier()

    def load_blk(buf, k):
        sl = pl.ds(k * IDXB, IDXB)
        pltpu.sync_copy(src3.at[w, sl], sblk[buf])
        pltpu.sync_copy(dst3.at[w, sl], dblk[buf])

    def g_desc(b, j):
        kb = (j // IDXB) % 2
        return pltpu.make_async_copy(
            m_hbm.at[sblk[kb].at[j % IDXB]], rows[b], sems[b])

    load_blk(0, 0)
    if NBLK > 1:
        load_blk(1, 1)
    for b in range(NBUF):
        g_desc(b, b).start()

    for j in range(C_PER_W):
        k = j // IDXB
        if j % IDXB == 0 and j >= IDXB and k + 1 < NBLK:
            load_blk((k + 1) % 2, k + 1)
        b = j % NBUF
        g_desc(b, j).wait()
        pltpu.sync_copy(rows[b], acc.at[dblk[k % 2].at[j % IDXB]], add=True)
        if j + NBUF < C_PER_W:
            g_desc(b, j + NBUF).start()

    plsc.subcore_barrier()
    sl = pl.ds(row0, ROWS_PER_TILE)
    pltpu.sync_copy(acc.at[sl], part.at[c, sl])


_edge_kernel = pl.kernel(
    _edge_body,
    out_type=jax.ShapeDtypeStruct((NC, N_PAD, D), jnp.float32),
    mesh=_mesh,
    scratch_types=[pltpu.VMEM((IDXB, CHUNK), jnp.int32)] * 4
    + [pltpu.VMEM((CHUNK, D), jnp.float32)] * NBUF + [
        pltpu.VMEM_SHARED((N_PAD, D), jnp.float32),
    ] + [pltpu.SemaphoreType.DMA] * NBUF,
)


def _norm_body(degp_ref, x_ref, onorm_ref, inorm_ref, m0_ref):
    ds_ = degp_ref[0, :, 0:1]
    dd_ = degp_ref[1, :, 0:1]
    mask = (lax.broadcasted_iota(jnp.int32, (N_PAD, 1), 0) < N).astype(
        jnp.float32)
    onorm = lax.rsqrt(jnp.maximum(ds_, 1.0)) * mask
    inorm = lax.rsqrt(jnp.maximum(dd_, 1.0))
    onorm_ref[...] = onorm
    inorm_ref[...] = inorm
    m0_ref[...] = x_ref[...] * onorm


def _norm_kernel(degp, x_pad):
    return pl.pallas_call(
        _norm_body,
        out_shape=(
            jax.ShapeDtypeStruct((N_PAD, 1), jnp.float32),
            jax.ShapeDtypeStruct((N_PAD, 1), jnp.float32),
            jax.ShapeDtypeStruct((N_PAD, D), jnp.float32),
        ),
    )(degp, x_pad)


def _layer_body(p_ref, inorm_ref, onorm_ref, w_ref, b_ref, mnext_ref):
    agg = (p_ref[0] + p_ref[1]) * inorm_ref[...]
    z = jnp.dot(agg, w_ref[...], preferred_element_type=jnp.float32)
    h = jnp.maximum(z + b_ref[...], 0.0)
    mnext_ref[...] = h * onorm_ref[...]


def _layer_kernel(part, inorm, onorm, w, b):
    return pl.pallas_call(
        _layer_body,
        out_shape=jax.ShapeDtypeStruct((N_PAD, D), jnp.float32),
    )(part, inorm, onorm, w, b)


def _final_body(p_ref, inorm_ref, w_ref, b_ref, wc_ref, bc_ref, out_ref):
    agg = (p_ref[0] + p_ref[1]) * inorm_ref[...]
    z = jnp.dot(agg, w_ref[...], preferred_element_type=jnp.float32)
    mask = (lax.broadcasted_iota(jnp.int32, (N_PAD, 1), 0) < N).astype(
        jnp.float32)
    h = jnp.maximum(z + b_ref[...], 0.0) * mask
    hg = jnp.sum(h, axis=0, keepdims=True) * (1.0 / N)
    out_ref[...] = (
        jnp.dot(hg, wc_ref[...], preferred_element_type=jnp.float32)
        + bc_ref[...])


def _final_kernel(part, inorm, w, b, wc, bc):
    return pl.pallas_call(
        _final_body,
        out_shape=jax.ShapeDtypeStruct((1, 2), jnp.float32),
    )(part, inorm, w, b, wc, bc)


@jax.jit
def kernel(x, edge_index, W0, b0, W1, b1, W2, b2, Wc, bc):
    pad = N + (jnp.arange(E_PAD - E, dtype=jnp.int32) % (N_PAD - N))
    src3 = jnp.concatenate([edge_index[0], pad]).reshape(NW, C_PER_W, CHUNK)
    dst3 = jnp.concatenate([edge_index[1], pad]).reshape(NW, C_PER_W, CHUNK)
    x_pad = jnp.concatenate(
        [x, jnp.zeros((N_PAD - N, D), jnp.float32)], axis=0)
    idxs = jnp.stack([src3.reshape(N_CHUNKS, CHUNK),
                      dst3.reshape(N_CHUNKS, CHUNK)])
    ones_rows = jnp.ones((CHUNK, D), jnp.float32)
    zeros_row = jnp.zeros((ROWS_PER_TILE, D), jnp.float32)

    degp = _deg_kernel(idxs, ones_rows, zeros_row)
    onorm, inorm, m = _norm_kernel(degp, x_pad)

    for (w, b) in ((W0, b0), (W1, b1)):
        part = _edge_kernel(m, src3, dst3, zeros_row)
        m = _layer_kernel(part, inorm, onorm, w, b.reshape(1, D))

    part = _edge_kernel(m, src3, dst3, zeros_row)
    return _final_kernel(part, inorm, W2, b2.reshape(1, D), Wc,
                         bc.reshape(1, 2))


# speedup vs baseline: 3.0701x; 1.0421x over previous
import functools

import jax
import jax.numpy as jnp
from jax import lax
from jax.experimental import pallas as pl
from jax.experimental.pallas import tpu as pltpu
from jax.experimental.pallas import tpu_sc as plsc

N = 10000
E = 320000
D = 128
NC = 2
NS = 16
NW = NC * NS
CHUNK = 128
NBUF = 2
C_PER_W = 80
IDXB = 16
NBLK = C_PER_W // IDXB
ECH = 64
EC_PER_W = (C_PER_W * CHUNK) // ECH
ENBUF = 4
EIDXB = 16
ENBLK = EC_PER_W // EIDXB
E_PAD = NW * C_PER_W * CHUNK
N_CHUNKS = E_PAD // CHUNK
C_PER_TILE = N_CHUNKS // NS
N_PAD = 10240
ROWS_PER_TILE = N_PAD // NS

_mesh = plsc.VectorSubcoreMesh(core_axis_name="c", subcore_axis_name="s")


def _deg_body(idxs, ones_hbm, zeros_hbm, degp, idx_v, ones_v, acc,
              sem0, sem1):
    sems = (sem0, sem1)
    c = lax.axis_index("c")
    s = lax.axis_index("s")
    row0 = s * ROWS_PER_TILE
    pltpu.sync_copy(zeros_hbm, acc.at[pl.ds(row0, ROWS_PER_TILE)])
    pltpu.sync_copy(ones_hbm, ones_v)
    pltpu.sync_copy(idxs.at[c, pl.ds(s * C_PER_TILE, C_PER_TILE)], idx_v)
    plsc.subcore_barrier()

    def s_desc(b, j):
        return pltpu.make_async_copy(ones_v, acc.at[idx_v.at[j]], sems[b])

    for b in range(2):
        s_desc(b, b).start(add=True)

    def body(g, carry):
        j = 2 * g
        for b in range(2):
            s_desc(b, j + b).wait()
            s_desc(b, j + b + 2).start(add=True)
        return carry

    lax.fori_loop(0, C_PER_TILE // 2 - 1, body, 0)
    for b in range(2):
        s_desc(b, C_PER_TILE - 2 + b).wait()
    plsc.subcore_barrier()
    sl = pl.ds(row0, ROWS_PER_TILE)
    pltpu.sync_copy(acc.at[sl], degp.at[c, sl])


_deg_kernel = pl.kernel(
    _deg_body,
    out_type=jax.ShapeDtypeStruct((NC, N_PAD, D), jnp.float32),
    mesh=_mesh,
    scratch_types=[
        pltpu.VMEM((C_PER_TILE, CHUNK), jnp.int32),
        pltpu.VMEM((CHUNK, D), jnp.float32),
        pltpu.VMEM_SHARED((N_PAD, D), jnp.float32),
        pltpu.SemaphoreType.DMA,
        pltpu.SemaphoreType.DMA,
    ],
)


def _edge_body(m_hbm, src3, dst3, zeros_hbm, part, *scr):
    sblk = scr[0:2]
    dblk = scr[2:4]
    rows = scr[4:4 + ENBUF]
    acc = scr[4 + ENBUF]
    sems = scr[5 + ENBUF:]
    c = lax.axis_index("c")
    s = lax.axis_index("s")
    w = c * NS + s
    row0 = s * ROWS_PER_TILE
    pltpu.sync_copy(zeros_hbm, acc.at[pl.ds(row0, ROWS_PER_TILE)])
    plsc.subcore_barrier()

    def load_blk(buf, k):
        sl = pl.ds(k * EIDXB, EIDXB)
        pltpu.sync_copy(src3.at[w, sl], sblk[buf])
        pltpu.sync_copy(dst3.at[w, sl], dblk[buf])

    def g_desc(b, j):
        kb = (j // EIDXB) % 2
        return pltpu.make_async_copy(
            m_hbm.at[sblk[kb].at[j % EIDXB]], rows[b], sems[b])

    load_blk(0, 0)
    if ENBLK > 1:
        load_blk(1, 1)
    for b in range(ENBUF):
        g_desc(b, b).start()

    for j in range(EC_PER_W):
        k = j // EIDXB
        if j % EIDXB == 0 and j >= EIDXB and k + 1 < ENBLK:
            load_blk((k + 1) % 2, k + 1)
        b = j % ENBUF
        g_desc(b, j).wait()
        pltpu.sync_copy(rows[b], acc.at[dblk[k % 2].at[j % EIDXB]], add=True)
        if j + ENBUF < EC_PER_W:
            g_desc(b, j + ENBUF).start()

    plsc.subcore_barrier()
    sl = pl.ds(row0, ROWS_PER_TILE)
    pltpu.sync_copy(acc.at[sl], part.at[c, sl])


_edge_kernel = pl.kernel(
    _edge_body,
    out_type=jax.ShapeDtypeStruct((NC, N_PAD, D), jnp.float32),
    mesh=_mesh,
    scratch_types=[pltpu.VMEM((EIDXB, ECH), jnp.int32)] * 4
    + [pltpu.VMEM((ECH, D), jnp.float32)] * ENBUF + [
        pltpu.VMEM_SHARED((N_PAD, D), jnp.float32),
    ] + [pltpu.SemaphoreType.DMA] * ENBUF,
)


def _norm_body(degp_ref, x_ref, onorm_ref, inorm_ref, m0_ref):
    ds_ = degp_ref[0, :, 0:1]
    dd_ = degp_ref[1, :, 0:1]
    mask = (lax.broadcasted_iota(jnp.int32, (N_PAD, 1), 0) < N).astype(
        jnp.float32)
    onorm = lax.rsqrt(jnp.maximum(ds_, 1.0)) * mask
    inorm = lax.rsqrt(jnp.maximum(dd_, 1.0))
    onorm_ref[...] = onorm
    inorm_ref[...] = inorm
    m0_ref[...] = x_ref[...] * onorm


def _norm_kernel(degp, x_pad):
    return pl.pallas_call(
        _norm_body,
        out_shape=(
            jax.ShapeDtypeStruct((N_PAD, 1), jnp.float32),
            jax.ShapeDtypeStruct((N_PAD, 1), jnp.float32),
            jax.ShapeDtypeStruct((N_PAD, D), jnp.float32),
        ),
    )(degp, x_pad)


def _layer_body(p_ref, inorm_ref, onorm_ref, w_ref, b_ref, mnext_ref):
    agg = (p_ref[0] + p_ref[1]) * inorm_ref[...]
    z = jnp.dot(agg, w_ref[...], preferred_element_type=jnp.float32)
    h = jnp.maximum(z + b_ref[...], 0.0)
    mnext_ref[...] = h * onorm_ref[...]


def _layer_kernel(part, inorm, onorm, w, b):
    return pl.pallas_call(
        _layer_body,
        out_shape=jax.ShapeDtypeStruct((N_PAD, D), jnp.float32),
    )(part, inorm, onorm, w, b)


def _final_body(p_ref, inorm_ref, w_ref, b_ref, wc_ref, bc_ref, out_ref):
    agg = (p_ref[0] + p_ref[1]) * inorm_ref[...]
    z = jnp.dot(agg, w_ref[...], preferred_element_type=jnp.float32)
    mask = (lax.broadcasted_iota(jnp.int32, (N_PAD, 1), 0) < N).astype(
        jnp.float32)
    h = jnp.maximum(z + b_ref[...], 0.0) * mask
    hg = jnp.sum(h, axis=0, keepdims=True) * (1.0 / N)
    out_ref[...] = (
        jnp.dot(hg, wc_ref[...], preferred_element_type=jnp.float32)
        + bc_ref[...])


def _final_kernel(part, inorm, w, b, wc, bc):
    return pl.pallas_call(
        _final_body,
        out_shape=jax.ShapeDtypeStruct((1, 2), jnp.float32),
    )(part, inorm, w, b, wc, bc)


@jax.jit
def kernel(x, edge_index, W0, b0, W1, b1, W2, b2, Wc, bc):
    pad = N + (jnp.arange(E_PAD - E, dtype=jnp.int32) % (N_PAD - N))
    src3 = jnp.concatenate([edge_index[0], pad]).reshape(NW, C_PER_W, CHUNK)
    dst3 = jnp.concatenate([edge_index[1], pad]).reshape(NW, C_PER_W, CHUNK)
    x_pad = jnp.concatenate(
        [x, jnp.zeros((N_PAD - N, D), jnp.float32)], axis=0)
    idxs = jnp.stack([src3.reshape(N_CHUNKS, CHUNK),
                      dst3.reshape(N_CHUNKS, CHUNK)])
    src3e = src3.reshape(NW, EC_PER_W, ECH)
    dst3e = dst3.reshape(NW, EC_PER_W, ECH)
    ones_rows = jnp.ones((CHUNK, D), jnp.float32)
    zeros_row = jnp.zeros((ROWS_PER_TILE, D), jnp.float32)

    degp = _deg_kernel(idxs, ones_rows, zeros_row)
    onorm, inorm, m = _norm_kernel(degp, x_pad)

    for (w, b) in ((W0, b0), (W1, b1)):
        part = _edge_kernel(m, src3e, dst3e, zeros_row)
        m = _layer_kernel(part, inorm, onorm, w, b.reshape(1, D))

    part = _edge_kernel(m, src3e, dst3e, zeros_row)
    return _final_kernel(part, inorm, W2, b2.reshape(1, D), Wc,
                         bc.reshape(1, 2))
